# baseline (speedup 1.0000x reference)
import dataclasses
import functools
import math

import jax
import jax.numpy as jnp
from jax import lax
from jax.experimental import pallas as pl
from jax.experimental.pallas import tpu as pltpu
from jax.experimental.pallas import tpu_sc as plsc

EPS = 1e-6
NEG = -1e9
_H = 16

_BT = 128


def _sc_compiler_params():
    cp = pltpu.CompilerParams()
    if "needs_layout_passes" in pltpu.CompilerParams.__dataclass_fields__:
        cp = dataclasses.replace(cp, needs_layout_passes=False)
    return cp


def _rms_matmul_kernel(x_ref, lnw_ref, w_ref, o_ref):
    x = x_ref[...]
    var = jnp.mean(x * x, axis=-1, keepdims=True)
    xn = x * lax.rsqrt(var + EPS) * lnw_ref[...]
    o_ref[...] = jnp.dot(xn.astype(jnp.bfloat16), w_ref[...],
                         preferred_element_type=jnp.float32)


def _rope_kernel(x_ref, cos_ref, sin_ref, o_ref):
    x = x_ref[0]
    hd = x.shape[-1]
    x1 = x[:, : hd // 2]
    x2 = x[:, hd // 2:]
    rot = jnp.concatenate([-x2, x1], axis=-1)
    o_ref[0] = x * cos_ref[...] + rot * sin_ref[...]


def _attn_kernel(q_ref, k_ref, v_ref, o_ref, acc_ref, m_ref, l_ref,
                 *, bq, bk, scale):
    iq = pl.program_id(1)
    ik = pl.program_id(2)

    @pl.when(ik == 0)
    def _init():
        acc_ref[...] = jnp.zeros_like(acc_ref)
        m_ref[...] = jnp.full_like(m_ref, NEG)
        l_ref[...] = jnp.zeros_like(l_ref)

    @pl.when(ik <= iq)
    def _blk():
        q = q_ref[0].astype(jnp.bfloat16)
        k = k_ref[0].astype(jnp.bfloat16)
        v = v_ref[0].astype(jnp.bfloat16)
        sc = lax.dot_general(q, k, (((1,), (1,)), ((), ())),
                             preferred_element_type=jnp.float32) * scale
        row = lax.broadcasted_iota(jnp.int32, sc.shape, 0) + iq * bq
        col = lax.broadcasted_iota(jnp.int32, sc.shape, 1) + ik * bk
        sc = sc + jnp.where(col > row, NEG, 0.0)
        m_old = m_ref[:, :1]
        m_new = jnp.maximum(m_old, jnp.max(sc, axis=1, keepdims=True))
        p = jnp.exp(sc - m_new)
        corr = jnp.exp(m_old - m_new)
        l_new = l_ref[:, :1] * corr + jnp.sum(p, axis=1, keepdims=True)
        pv = lax.dot_general(p.astype(jnp.bfloat16), v,
                             (((1,), (0,)), ((), ())),
                             preferred_element_type=jnp.float32)
        acc_ref[...] = acc_ref[...] * corr + pv
        m_ref[...] = jnp.broadcast_to(m_new, m_ref.shape)
        l_ref[...] = jnp.broadcast_to(l_new, l_ref.shape)

        @pl.when(ik == iq)
        def _final():
            o_ref[0] = acc_ref[...] / l_ref[:, :1]


def _mm_add_kernel(a_ref, b_ref, r_ref, o_ref):
    o_ref[...] = r_ref[...] + jnp.dot(a_ref[...].astype(jnp.bfloat16),
                                      b_ref[...].astype(jnp.bfloat16),
                                      preferred_element_type=jnp.float32)


def _rms_kernel(x_ref, w_ref, o_ref):
    x = x_ref[...]
    var = jnp.mean(x * x, axis=-1, keepdims=True)
    o_ref[...] = x * lax.rsqrt(var + EPS) * w_ref[...]


def _router_kernel(x_ref, wg_ref, logits_ref, fullw_ref, idx_ref, topw_ref, *, e):
    x = x_ref[...]
    logits = jnp.dot(x, wg_ref[...], preferred_element_type=jnp.float32)
    logits_ref[...] = logits
    lane = lax.broadcasted_iota(jnp.int32, logits.shape, 1)
    valid = lane < e
    ml = jnp.where(valid, logits, NEG)
    mx = jnp.max(ml, axis=1, keepdims=True)
    ex = jnp.where(valid, jnp.exp(ml - mx), 0.0)
    probs = ex / jnp.sum(ex, axis=1, keepdims=True)
    m1 = jnp.max(probs, axis=1, keepdims=True)
    i1 = jnp.min(jnp.where(probs == m1, lane, e), axis=1, keepdims=True)
    p2 = jnp.where(lane == i1, -1.0, probs)
    m2 = jnp.max(p2, axis=1, keepdims=True)
    i2 = jnp.min(jnp.where(p2 == m2, lane, e), axis=1, keepdims=True)
    tot = m1 + m2
    w1n = m1 / tot
    w2n = m2 / tot
    fullw_ref[...] = (jnp.where(lane == i1, w1n, 0.0)
                      + jnp.where(lane == i2, w2n, 0.0))
    idx_ref[...] = jnp.where(lane == 0, i1, jnp.where(lane == 1, i2, 0))
    topw_ref[...] = jnp.where(lane == 0, w1n, jnp.where(lane == 1, w2n, 0.0))


def _sc_dispatch(ids, wflat, t_tokens, nslot, ntp):
    a_n = ids.shape[0]
    nchunk = a_n // 16
    mesh = plsc.VectorSubcoreMesh(core_axis_name="c", subcore_axis_name="s")

    @functools.partial(
        pl.kernel,
        out_type=[
            jax.ShapeDtypeStruct((nslot,), jnp.int32),
            jax.ShapeDtypeStruct((nslot,), jnp.float32),
            jax.ShapeDtypeStruct((t_tokens,), jnp.int32),
            jax.ShapeDtypeStruct((t_tokens,), jnp.int32),
            jax.ShapeDtypeStruct((ntp,), jnp.int32),
            jax.ShapeDtypeStruct((ntp,), jnp.int32),
        ],
        mesh=mesh,
        scratch_types=[
            pltpu.VMEM((a_n,), jnp.int32),
            pltpu.VMEM((a_n,), jnp.float32),
            pltpu.VMEM((nslot,), jnp.int32),
            pltpu.VMEM((nslot,), jnp.float32),
            pltpu.VMEM((t_tokens,), jnp.int32),
            pltpu.VMEM((t_tokens,), jnp.int32),
            pltpu.VMEM((ntp,), jnp.int32),
            pltpu.VMEM((ntp,), jnp.int32),
            pltpu.VMEM((16,), jnp.int32),
            pltpu.VMEM((16,), jnp.int32),
        ],
        compiler_params=_sc_compiler_params(),
    )
    def disp(ids_hbm, w_hbm, st_hbm, sw_hbm, ia_hbm, ib_hbm, te_hbm, tv_hbm,
             ids_v, w_v, st_v, sw_v, ia_v, ib_v, te_v, tv_v, ends_v, cnt_v):
        @pl.when((lax.axis_index("c") == 0) & (lax.axis_index("s") == 0))
        def _():
            pltpu.sync_copy(ids_hbm, ids_v)
            pltpu.sync_copy(w_hbm, w_v)
            iota = lax.iota(jnp.int32, 16)
            zeros16 = jnp.zeros((16,), jnp.int32)
            ends_v[...] = zeros16

            @pl.loop(0, nchunk)
            def _hist(c):
                vec = ids_v[pl.ds(c * 16, 16)]
                hv = ends_v[...]
                for e in range(8):
                    ce = jnp.sum((vec == e).astype(jnp.int32))
                    hv = hv + jnp.where(iota == e, ce, 0)
                ends_v[...] = hv

            h16 = ends_v[...]
            hp = ((h16 + (_BT - 1)) // _BT) * _BT
            ends = plsc.cumsum(hp)
            cnt_v[...] = ends - hp

            total_tiles = jnp.sum(jnp.where(iota == 7, ends, 0)) // _BT
            for c3 in range(ntp // 16):
                tid = iota + 16 * c3
                acc = jnp.zeros((16,), jnp.int32)
                for e in range(8):
                    ends_e = jnp.sum(jnp.where(iota == e, ends, 0))
                    acc = acc + (tid * _BT >= ends_e).astype(jnp.int32)
                te_v[pl.ds(16 * c3, 16)] = jnp.minimum(acc, 7)
                tv_v[pl.ds(16 * c3, 16)] = (tid < total_tiles).astype(jnp.int32)

            @pl.loop(0, nslot // 16)
            def _zero(i):
                st_v[pl.ds(i * 16, 16)] = zeros16
                sw_v[pl.ds(i * 16, 16)] = jnp.zeros((16,), jnp.float32)

            @pl.loop(0, nchunk)
            def _place(c):
                base = c * 16
                vec = ids_v[pl.ds(base, 16)]
                wv = w_v[pl.ds(base, 16)]
                cvec = cnt_v[...]
                rank = jnp.zeros((16,), jnp.int32)
                bse = jnp.zeros((16,), jnp.int32)
                for e in range(8):
                    m = vec == e
                    mi = m.astype(jnp.int32)
                    cs = plsc.cumsum(mi)
                    rank = jnp.where(m, cs - 1, rank)
                    ce = jnp.sum(jnp.where(iota == e, cvec, 0))
                    bse = jnp.where(m, ce, bse)
                    cvec = cvec + jnp.where(iota == e, jnp.sum(mi), 0)
                cnt_v[...] = cvec
                slot = bse + rank
                toks = (base + iota) // 2
                plsc.store_scatter(st_v, [slot], toks)
                plsc.store_scatter(sw_v, [slot], wv)
                evm = (iota % 2) == 0
                plsc.store_scatter(ia_v, [toks], slot, mask=evm)
                plsc.store_scatter(ib_v, [toks], slot,
                                   mask=jnp.logical_not(evm))

            pltpu.sync_copy(st_v, st_hbm)
            pltpu.sync_copy(sw_v, sw_hbm)
            pltpu.sync_copy(ia_v, ia_hbm)
            pltpu.sync_copy(ib_v, ib_hbm)
            pltpu.sync_copy(te_v, te_hbm)
            pltpu.sync_copy(tv_v, tv_hbm)

    return disp(ids, wflat)


def _sc_gather_rows(x2, sorted_tok, nslot, d):
    mesh = plsc.VectorSubcoreMesh(core_axis_name="c", subcore_axis_name="s")
    win = 32
    nworker = 32
    per = nslot // nworker

    @functools.partial(
        pl.kernel,
        out_type=jax.ShapeDtypeStruct((nslot, d), jnp.float32),
        mesh=mesh,
        scratch_types=[
            pltpu.VMEM((nslot,), jnp.int32),
            pltpu.VMEM((win, d), jnp.float32),
            pltpu.SemaphoreType.DMA,
        ],
        compiler_params=_sc_compiler_params(),
    )
    def gat(x_hbm, i_hbm, o_hbm, idx_v, rows_v, sem):
        wid = lax.axis_index("s") * 2 + lax.axis_index("c")
        pltpu.sync_copy(i_hbm, idx_v)
        base = wid * per

        @pl.loop(0, per // win)
        def _(w):
            off = base + w * win
            pltpu.async_copy(x_hbm.at[idx_v.at[pl.ds(off, win)]],
                             rows_v, sem).wait()
            pltpu.sync_copy(rows_v, o_hbm.at[pl.ds(off, win)])

    return gat(x2, sorted_tok)


def _sc_combine(h, yw, inva, invb, t_tokens, d):
    mesh = plsc.VectorSubcoreMesh(core_axis_name="c", subcore_axis_name="s")
    win = 16
    nworker = 32
    per = t_tokens // nworker
    nw = per // win

    @functools.partial(
        pl.kernel,
        out_type=jax.ShapeDtypeStruct((t_tokens, d), jnp.float32),
        mesh=mesh,
        scratch_types=[
            pltpu.VMEM((per,), jnp.int32),
            pltpu.VMEM((per,), jnp.int32),
            pltpu.VMEM((2, win, d), jnp.float32),
            pltpu.VMEM((2, win, d), jnp.float32),
            pltpu.VMEM((win, d), jnp.float32),
            pltpu.SemaphoreType.DMA,
            pltpu.SemaphoreType.DMA,
        ],
        compiler_params=_sc_compiler_params(),
    )
    def comb(yw_hbm, ia_hbm, ib_hbm, h_hbm, o_hbm,
             ia_v, ib_v, ya_s, yb_s, hb_s, sem0, sem1):
        wid = lax.axis_index("s") * 2 + lax.axis_index("c")
        t0 = wid * per
        pltpu.sync_copy(ia_hbm.at[pl.ds(t0, per)], ia_v)
        pltpu.sync_copy(ib_hbm.at[pl.ds(t0, per)], ib_v)
        sems = (sem0, sem1)

        def start(w, buf):
            pltpu.async_copy(yw_hbm.at[ia_v.at[pl.ds(w * win, win)]],
                             ya_s.at[buf], sems[buf])
            pltpu.async_copy(yw_hbm.at[ib_v.at[pl.ds(w * win, win)]],
                             yb_s.at[buf], sems[buf])

        def finish(w, buf):
            pltpu.make_async_copy(yw_hbm.at[ia_v.at[pl.ds(w * win, win)]],
                                  ya_s.at[buf], sems[buf]).wait()
            pltpu.make_async_copy(yw_hbm.at[ib_v.at[pl.ds(w * win, win)]],
                                  yb_s.at[buf], sems[buf]).wait()

        start(0, 0)
        for w in range(nw):
            buf = w % 2
            if w + 1 < nw:
                start(w + 1, 1 - buf)
            tb = t0 + w * win
            pltpu.sync_copy(h_hbm.at[pl.ds(tb, win)], hb_s)
            finish(w, buf)

            @pl.loop(0, win)
            def _rows(r):
                for cc in range(d // 16):
                    sl = pl.ds(cc * 16, 16)
                    hb_s[r, sl] = (hb_s[r, sl] + ya_s[buf, r, sl]
                                   + yb_s[buf, r, sl])

            pltpu.sync_copy(hb_s, o_hbm.at[pl.ds(tb, win)])

    return comb(yw, inva, invb, h)


def _moe_h_kernel(te_ref, tv_ref, x_ref, w1_ref, w3_ref, h_ref):
    t = pl.program_id(1)

    @pl.when(tv_ref[t] == 1)
    def _():
        x = x_ref[...].astype(jnp.bfloat16)
        a = jnp.dot(x, w1_ref[0].astype(jnp.bfloat16),
                    preferred_element_type=jnp.float32)
        b3 = jnp.dot(x, w3_ref[0].astype(jnp.bfloat16),
                     preferred_element_type=jnp.float32)
        sil = a / (1.0 + jnp.exp(-a))
        h_ref[...] = (sil * b3).astype(jnp.bfloat16)


def _moe_y_kernel(te_ref, tv_ref, h_ref, w2_ref, sw_ref, y_ref):
    t = pl.program_id(0)

    @pl.when(tv_ref[t] == 1)
    def _():
        y = jnp.dot(h_ref[...], w2_ref[0].astype(jnp.bfloat16),
                    preferred_element_type=jnp.float32)
        y_ref[...] = y * sw_ref[...][:, 0:1]

    @pl.when(tv_ref[t] == 0)
    def _():
        y_ref[...] = jnp.zeros_like(y_ref)


def kernel(hidden_states, attention_mask, position_ids, ln1_w, ln2_w,
           Wq, Wk, Wv, Wo, Wg, w1, w2, w3):
    b, s, d = hidden_states.shape
    heads = _H
    hd = d // heads
    ne = Wg.shape[1]
    dff = w1.shape[2]
    scale = 1.0 / math.sqrt(hd)

    bm = min(s, 512)
    bq = min(s, 512)
    bn = min(d, 512)
    brr = min(s, 256)
    bf = 1408 if dff % 1408 == 0 else dff
    nf = dff // bf

    x0 = hidden_states.reshape(s, d)
    ln1 = ln1_w.reshape(1, d)
    ln2 = ln2_w.reshape(1, d)
    wqkv = jnp.concatenate([Wq, Wk, Wv], axis=1).astype(jnp.bfloat16)

    inv_freq = 1.0 / (10000.0 ** (jnp.arange(0, hd, 2, dtype=jnp.float32) / hd))
    t = jnp.arange(s, dtype=jnp.float32)
    freqs = jnp.outer(t, inv_freq)
    emb = jnp.concatenate([freqs, freqs], axis=-1)
    cos = jnp.cos(emb)[position_ids[0]]
    sin = jnp.sin(emb)[position_ids[0]]

    qkv = pl.pallas_call(
        _rms_matmul_kernel,
        grid=(s // bm, (3 * d) // bn),
        in_specs=[
            pl.BlockSpec((bm, d), lambda i, j: (i, 0)),
            pl.BlockSpec((1, d), lambda i, j: (0, 0)),
            pl.BlockSpec((d, bn), lambda i, j: (0, j)),
        ],
        out_specs=pl.BlockSpec((bm, bn), lambda i, j: (i, j)),
        out_shape=jax.ShapeDtypeStruct((s, 3 * d), jnp.float32),
    )(x0, ln1, wqkv)

    qkv3 = qkv.reshape(s, 3 * heads, hd).transpose(1, 0, 2)
    qk3 = qkv3[: 2 * heads]
    v3 = qkv3[2 * heads:]
    roped = pl.pallas_call(
        _rope_kernel,
        grid=(2 * heads,),
        in_specs=[
            pl.BlockSpec((1, s, hd), lambda h: (h, 0, 0)),
            pl.BlockSpec((s, hd), lambda h: (0, 0)),
            pl.BlockSpec((s, hd), lambda h: (0, 0)),
        ],
        out_specs=pl.BlockSpec((1, s, hd), lambda h: (h, 0, 0)),
        out_shape=jax.ShapeDtypeStruct((2 * heads, s, hd), jnp.float32),
    )(qk3, cos, sin)

    bk = bq
    attno = pl.pallas_call(
        functools.partial(_attn_kernel, bq=bq, bk=bk, scale=scale),
        grid=(heads, s // bq, s // bk),
        in_specs=[
            pl.BlockSpec((1, bq, hd), lambda h, iq, ik: (h, iq, 0)),
            pl.BlockSpec((1, bk, hd), lambda h, iq, ik: (heads + h, ik, 0)),
            pl.BlockSpec((1, bk, hd), lambda h, iq, ik: (h, ik, 0)),
        ],
        out_specs=pl.BlockSpec((1, bq, hd), lambda h, iq, ik: (h, iq, 0)),
        out_shape=jax.ShapeDtypeStruct((heads, s, hd), jnp.float32),
        scratch_shapes=[
            pltpu.VMEM((bq, hd), jnp.float32),
            pltpu.VMEM((bq, 128), jnp.float32),
            pltpu.VMEM((bq, 128), jnp.float32),
        ],
    )(roped, roped, v3)
    attno2 = attno.transpose(1, 0, 2).reshape(s, d)

    h = pl.pallas_call(
        _mm_add_kernel,
        grid=(s // bm, d // bn),
        in_specs=[
            pl.BlockSpec((bm, d), lambda i, j: (i, 0)),
            pl.BlockSpec((d, bn), lambda i, j: (0, j)),
            pl.BlockSpec((bm, bn), lambda i, j: (i, j)),
        ],
        out_specs=pl.BlockSpec((bm, bn), lambda i, j: (i, j)),
        out_shape=jax.ShapeDtypeStruct((s, d), jnp.float32),
    )(attno2, Wo, x0)

    x2 = pl.pallas_call(
        _rms_kernel,
        grid=(s // bm,),
        in_specs=[
            pl.BlockSpec((bm, d), lambda i: (i, 0)),
            pl.BlockSpec((1, d), lambda i: (0, 0)),
        ],
        out_specs=pl.BlockSpec((bm, d), lambda i: (i, 0)),
        out_shape=jax.ShapeDtypeStruct((s, d), jnp.float32),
    )(h, ln2)

    wg_pad = jnp.pad(Wg, ((0, 0), (0, 128 - ne)))
    logits_p, fullw, top_idx, top_w = pl.pallas_call(
        functools.partial(_router_kernel, e=ne),
        grid=(s // bm,),
        in_specs=[
            pl.BlockSpec((bm, d), lambda i: (i, 0)),
            pl.BlockSpec((d, 128), lambda i: (0, 0)),
        ],
        out_specs=[
            pl.BlockSpec((bm, 128), lambda i: (i, 0)),
            pl.BlockSpec((bm, 128), lambda i: (i, 0)),
            pl.BlockSpec((bm, 128), lambda i: (i, 0)),
            pl.BlockSpec((bm, 128), lambda i: (i, 0)),
        ],
        out_shape=[
            jax.ShapeDtypeStruct((s, 128), jnp.float32),
            jax.ShapeDtypeStruct((s, 128), jnp.float32),
            jax.ShapeDtypeStruct((s, 128), jnp.int32),
            jax.ShapeDtypeStruct((s, 128), jnp.float32),
        ],
    )(x2, wg_pad)

    a_n = 2 * s
    nslot = a_n + ne * _BT
    nt = nslot // _BT
    ntp = ((nt + 15) // 16) * 16
    ids_flat = top_idx[:, :2].reshape(-1)
    w_flat = top_w[:, :2].reshape(-1)
    st, sw, inva, invb, te, tv = _sc_dispatch(ids_flat, w_flat, s, nslot, ntp)

    xs = _sc_gather_rows(x2, st, nslot, d)

    grid_a = pltpu.PrefetchScalarGridSpec(
        num_scalar_prefetch=2,
        grid=(nf, nt),
        in_specs=[
            pl.BlockSpec((_BT, d), lambda f, t, te_r, tv_r: (t, 0)),
            pl.BlockSpec((1, d, bf), lambda f, t, te_r, tv_r: (te_r[t], 0, f)),
            pl.BlockSpec((1, d, bf), lambda f, t, te_r, tv_r: (te_r[t], 0, f)),
        ],
        out_specs=pl.BlockSpec((_BT, bf), lambda f, t, te_r, tv_r: (t, f)),
    )
    hbuf = pl.pallas_call(
        _moe_h_kernel, grid_spec=grid_a,
        out_shape=jax.ShapeDtypeStruct((nslot, dff), jnp.bfloat16),
    )(te, tv, xs, w1, w3)

    swb = jnp.broadcast_to(sw[:, None], (nslot, 128))
    grid_b = pltpu.PrefetchScalarGridSpec(
        num_scalar_prefetch=2,
        grid=(nt,),
        in_specs=[
            pl.BlockSpec((_BT, dff), lambda t, te_r, tv_r: (t, 0)),
            pl.BlockSpec((1, dff, d), lambda t, te_r, tv_r: (te_r[t], 0, 0)),
            pl.BlockSpec((_BT, 128), lambda t, te_r, tv_r: (t, 0)),
        ],
        out_specs=pl.BlockSpec((_BT, d), lambda t, te_r, tv_r: (t, 0)),
    )
    yw = pl.pallas_call(
        _moe_y_kernel, grid_spec=grid_b,
        out_shape=jax.ShapeDtypeStruct((nslot, d), jnp.float32),
    )(te, tv, hbuf, w2, swb)

    out2d = _sc_combine(h, yw, inva, invb, s, d)

    return (out2d.reshape(b, s, d), logits_p[:, :ne])


# speedup vs baseline: 1.0262x; 1.0262x over previous
import dataclasses
import functools
import math

import jax
import jax.numpy as jnp
from jax import lax
from jax.experimental import pallas as pl
from jax.experimental.pallas import tpu as pltpu
from jax.experimental.pallas import tpu_sc as plsc

EPS = 1e-6
NEG = -1e9
_H = 16

_BT = 128


def _sc_compiler_params():
    cp = pltpu.CompilerParams()
    if "needs_layout_passes" in pltpu.CompilerParams.__dataclass_fields__:
        cp = dataclasses.replace(cp, needs_layout_passes=False)
    return cp


def _rms_matmul_kernel(x_ref, lnw_ref, w_ref, o_ref):
    x = x_ref[...]
    var = jnp.mean(x * x, axis=-1, keepdims=True)
    xn = x * lax.rsqrt(var + EPS) * lnw_ref[...]
    o_ref[...] = jnp.dot(xn.astype(jnp.bfloat16), w_ref[...],
                         preferred_element_type=jnp.float32)


def _rope_kernel(x_ref, cos_ref, sin_ref, o_ref):
    x = x_ref[0]
    hd = x.shape[-1]
    x1 = x[:, : hd // 2]
    x2 = x[:, hd // 2:]
    rot = jnp.concatenate([-x2, x1], axis=-1)
    o_ref[0] = x * cos_ref[...] + rot * sin_ref[...]


def _attn_kernel(q_ref, k_ref, v_ref, o_ref, acc_ref, m_ref, l_ref,
                 *, bq, bk, scale):
    iq = pl.program_id(1)
    ik = pl.program_id(2)

    @pl.when(ik == 0)
    def _init():
        acc_ref[...] = jnp.zeros_like(acc_ref)
        m_ref[...] = jnp.full_like(m_ref, NEG)
        l_ref[...] = jnp.zeros_like(l_ref)

    @pl.when(ik <= iq)
    def _blk():
        q = q_ref[0].astype(jnp.bfloat16)
        k = k_ref[0, pl.ds(ik * bk, bk), :].astype(jnp.bfloat16)
        v = v_ref[0, pl.ds(ik * bk, bk), :].astype(jnp.bfloat16)
        sc = lax.dot_general(q, k, (((1,), (1,)), ((), ())),
                             preferred_element_type=jnp.float32) * scale
        row = lax.broadcasted_iota(jnp.int32, sc.shape, 0) + iq * bq
        col = lax.broadcasted_iota(jnp.int32, sc.shape, 1) + ik * bk
        sc = sc + jnp.where(col > row, NEG, 0.0)
        m_old = m_ref[:, :1]
        m_new = jnp.maximum(m_old, jnp.max(sc, axis=1, keepdims=True))
        p = jnp.exp(sc - m_new)
        corr = jnp.exp(m_old - m_new)
        l_new = l_ref[:, :1] * corr + jnp.sum(p, axis=1, keepdims=True)
        pv = lax.dot_general(p.astype(jnp.bfloat16), v,
                             (((1,), (0,)), ((), ())),
                             preferred_element_type=jnp.float32)
        acc_ref[...] = acc_ref[...] * corr + pv
        m_ref[...] = jnp.broadcast_to(m_new, m_ref.shape)
        l_ref[...] = jnp.broadcast_to(l_new, l_ref.shape)

        @pl.when(ik == iq)
        def _final():
            o_ref[0] = acc_ref[...] / l_ref[:, :1]


def _mm_add_kernel(a_ref, b_ref, r_ref, o_ref):
    o_ref[...] = r_ref[...] + jnp.dot(a_ref[...].astype(jnp.bfloat16),
                                      b_ref[...].astype(jnp.bfloat16),
                                      preferred_element_type=jnp.float32)


def _rms_kernel(x_ref, w_ref, o_ref):
    x = x_ref[...]
    var = jnp.mean(x * x, axis=-1, keepdims=True)
    o_ref[...] = x * lax.rsqrt(var + EPS) * w_ref[...]


def _router_kernel(x_ref, wg_ref, logits_ref, fullw_ref, idx_ref, topw_ref, *, e):
    x = x_ref[...]
    logits = jnp.dot(x, wg_ref[...], preferred_element_type=jnp.float32)
    logits_ref[...] = logits
    lane = lax.broadcasted_iota(jnp.int32, logits.shape, 1)
    valid = lane < e
    ml = jnp.where(valid, logits, NEG)
    mx = jnp.max(ml, axis=1, keepdims=True)
    ex = jnp.where(valid, jnp.exp(ml - mx), 0.0)
    probs = ex / jnp.sum(ex, axis=1, keepdims=True)
    m1 = jnp.max(probs, axis=1, keepdims=True)
    i1 = jnp.min(jnp.where(probs == m1, lane, e), axis=1, keepdims=True)
    p2 = jnp.where(lane == i1, -1.0, probs)
    m2 = jnp.max(p2, axis=1, keepdims=True)
    i2 = jnp.min(jnp.where(p2 == m2, lane, e), axis=1, keepdims=True)
    tot = m1 + m2
    w1n = m1 / tot
    w2n = m2 / tot
    fullw_ref[...] = (jnp.where(lane == i1, w1n, 0.0)
                      + jnp.where(lane == i2, w2n, 0.0))
    idx_ref[...] = jnp.where(lane == 0, i1, jnp.where(lane == 1, i2, 0))
    topw_ref[...] = jnp.where(lane == 0, w1n, jnp.where(lane == 1, w2n, 0.0))


def _sc_dispatch(ids, wflat, t_tokens, nslot, ntp):
    a_n = ids.shape[0]
    nchunk = a_n // 16
    mesh = plsc.VectorSubcoreMesh(core_axis_name="c", subcore_axis_name="s")

    @functools.partial(
        pl.kernel,
        out_type=[
            jax.ShapeDtypeStruct((nslot,), jnp.int32),
            jax.ShapeDtypeStruct((nslot,), jnp.float32),
            jax.ShapeDtypeStruct((t_tokens,), jnp.int32),
            jax.ShapeDtypeStruct((t_tokens,), jnp.int32),
            jax.ShapeDtypeStruct((ntp,), jnp.int32),
            jax.ShapeDtypeStruct((ntp,), jnp.int32),
        ],
        mesh=mesh,
        scratch_types=[
            pltpu.VMEM((a_n,), jnp.int32),
            pltpu.VMEM((a_n,), jnp.float32),
            pltpu.VMEM((nslot,), jnp.int32),
            pltpu.VMEM((nslot,), jnp.float32),
            pltpu.VMEM((t_tokens,), jnp.int32),
            pltpu.VMEM((t_tokens,), jnp.int32),
            pltpu.VMEM((ntp,), jnp.int32),
            pltpu.VMEM((ntp,), jnp.int32),
            pltpu.VMEM((16,), jnp.int32),
            pltpu.VMEM((16,), jnp.int32),
        ],
        compiler_params=_sc_compiler_params(),
    )
    def disp(ids_hbm, w_hbm, st_hbm, sw_hbm, ia_hbm, ib_hbm, te_hbm, tv_hbm,
             ids_v, w_v, st_v, sw_v, ia_v, ib_v, te_v, tv_v, ends_v, cnt_v):
        @pl.when((lax.axis_index("c") == 0) & (lax.axis_index("s") == 0))
        def _():
            pltpu.sync_copy(ids_hbm, ids_v)
            pltpu.sync_copy(w_hbm, w_v)
            iota = lax.iota(jnp.int32, 16)
            zeros16 = jnp.zeros((16,), jnp.int32)
            ends_v[...] = zeros16

            @pl.loop(0, nchunk)
            def _hist(c):
                vec = ids_v[pl.ds(c * 16, 16)]
                hv = ends_v[...]
                for e in range(8):
                    ce = jnp.sum((vec == e).astype(jnp.int32))
                    hv = hv + jnp.where(iota == e, ce, 0)
                ends_v[...] = hv

            h16 = ends_v[...]
            hp = ((h16 + (_BT - 1)) // _BT) * _BT
            ends = plsc.cumsum(hp)
            cnt_v[...] = ends - hp

            total_tiles = jnp.sum(jnp.where(iota == 7, ends, 0)) // _BT
            for c3 in range(ntp // 16):
                tid = iota + 16 * c3
                acc = jnp.zeros((16,), jnp.int32)
                for e in range(8):
                    ends_e = jnp.sum(jnp.where(iota == e, ends, 0))
                    acc = acc + (tid * _BT >= ends_e).astype(jnp.int32)
                te_v[pl.ds(16 * c3, 16)] = jnp.minimum(acc, 7)
                tv_v[pl.ds(16 * c3, 16)] = (tid < total_tiles).astype(jnp.int32)

            @pl.loop(0, nslot // 16)
            def _zero(i):
                st_v[pl.ds(i * 16, 16)] = zeros16
                sw_v[pl.ds(i * 16, 16)] = jnp.zeros((16,), jnp.float32)

            @pl.loop(0, nchunk)
            def _place(c):
                base = c * 16
                vec = ids_v[pl.ds(base, 16)]
                wv = w_v[pl.ds(base, 16)]
                cvec = cnt_v[...]
                rank = jnp.zeros((16,), jnp.int32)
                bse = jnp.zeros((16,), jnp.int32)
                for e in range(8):
                    m = vec == e
                    mi = m.astype(jnp.int32)
                    cs = plsc.cumsum(mi)
                    rank = jnp.where(m, cs - 1, rank)
                    ce = jnp.sum(jnp.where(iota == e, cvec, 0))
                    bse = jnp.where(m, ce, bse)
                    cvec = cvec + jnp.where(iota == e, jnp.sum(mi), 0)
                cnt_v[...] = cvec
                slot = bse + rank
                toks = (base + iota) // 2
                plsc.store_scatter(st_v, [slot], toks)
                plsc.store_scatter(sw_v, [slot], wv)
                evm = (iota % 2) == 0
                plsc.store_scatter(ia_v, [toks], slot, mask=evm)
                plsc.store_scatter(ib_v, [toks], slot,
                                   mask=jnp.logical_not(evm))

            pltpu.sync_copy(st_v, st_hbm)
            pltpu.sync_copy(sw_v, sw_hbm)
            pltpu.sync_copy(ia_v, ia_hbm)
            pltpu.sync_copy(ib_v, ib_hbm)
            pltpu.sync_copy(te_v, te_hbm)
            pltpu.sync_copy(tv_v, tv_hbm)

    return disp(ids, wflat)


def _sc_gather_rows(x2, sorted_tok, nslot, d):
    mesh = plsc.VectorSubcoreMesh(core_axis_name="c", subcore_axis_name="s")
    win = 32
    nworker = 32
    per = nslot // nworker

    @functools.partial(
        pl.kernel,
        out_type=jax.ShapeDtypeStruct((nslot, d), jnp.float32),
        mesh=mesh,
        scratch_types=[
            pltpu.VMEM((nslot,), jnp.int32),
            pltpu.VMEM((win, d), jnp.float32),
            pltpu.SemaphoreType.DMA,
        ],
        compiler_params=_sc_compiler_params(),
    )
    def gat(x_hbm, i_hbm, o_hbm, idx_v, rows_v, sem):
        wid = lax.axis_index("s") * 2 + lax.axis_index("c")
        pltpu.sync_copy(i_hbm, idx_v)
        base = wid * per

        @pl.loop(0, per // win)
        def _(w):
            off = base + w * win
            pltpu.async_copy(x_hbm.at[idx_v.at[pl.ds(off, win)]],
                             rows_v, sem).wait()
            pltpu.sync_copy(rows_v, o_hbm.at[pl.ds(off, win)])

    return gat(x2, sorted_tok)


def _sc_combine(h, yw, inva, invb, t_tokens, d):
    mesh = plsc.VectorSubcoreMesh(core_axis_name="c", subcore_axis_name="s")
    win = 16
    nworker = 32
    per = t_tokens // nworker
    nw = per // win

    @functools.partial(
        pl.kernel,
        out_type=jax.ShapeDtypeStruct((t_tokens, d), jnp.float32),
        mesh=mesh,
        scratch_types=[
            pltpu.VMEM((per,), jnp.int32),
            pltpu.VMEM((per,), jnp.int32),
            pltpu.VMEM((2, win, d), jnp.float32),
            pltpu.VMEM((2, win, d), jnp.float32),
            pltpu.VMEM((win, d), jnp.float32),
            pltpu.SemaphoreType.DMA,
            pltpu.SemaphoreType.DMA,
        ],
        compiler_params=_sc_compiler_params(),
    )
    def comb(yw_hbm, ia_hbm, ib_hbm, h_hbm, o_hbm,
             ia_v, ib_v, ya_s, yb_s, hb_s, sem0, sem1):
        wid = lax.axis_index("s") * 2 + lax.axis_index("c")
        t0 = wid * per
        pltpu.sync_copy(ia_hbm.at[pl.ds(t0, per)], ia_v)
        pltpu.sync_copy(ib_hbm.at[pl.ds(t0, per)], ib_v)
        sems = (sem0, sem1)

        def start(w, buf):
            pltpu.async_copy(yw_hbm.at[ia_v.at[pl.ds(w * win, win)]],
                             ya_s.at[buf], sems[buf])
            pltpu.async_copy(yw_hbm.at[ib_v.at[pl.ds(w * win, win)]],
                             yb_s.at[buf], sems[buf])

        def finish(w, buf):
            pltpu.make_async_copy(yw_hbm.at[ia_v.at[pl.ds(w * win, win)]],
                                  ya_s.at[buf], sems[buf]).wait()
            pltpu.make_async_copy(yw_hbm.at[ib_v.at[pl.ds(w * win, win)]],
                                  yb_s.at[buf], sems[buf]).wait()

        start(0, 0)
        for w in range(nw):
            buf = w % 2
            if w + 1 < nw:
                start(w + 1, 1 - buf)
            tb = t0 + w * win
            pltpu.sync_copy(h_hbm.at[pl.ds(tb, win)], hb_s)
            finish(w, buf)

            @pl.loop(0, win)
            def _rows(r):
                for cc in range(d // 16):
                    sl = pl.ds(cc * 16, 16)
                    hb_s[r, sl] = (hb_s[r, sl] + ya_s[buf, r, sl]
                                   + yb_s[buf, r, sl])

            pltpu.sync_copy(hb_s, o_hbm.at[pl.ds(tb, win)])

    return comb(yw, inva, invb, h)


def _moe_h_kernel(te_ref, tv_ref, x_ref, w1_ref, w3_ref, h_ref):
    t = pl.program_id(1)

    @pl.when(tv_ref[t] == 1)
    def _():
        x = x_ref[...].astype(jnp.bfloat16)
        a = jnp.dot(x, w1_ref[0].astype(jnp.bfloat16),
                    preferred_element_type=jnp.float32)
        b3 = jnp.dot(x, w3_ref[0].astype(jnp.bfloat16),
                     preferred_element_type=jnp.float32)
        sil = a / (1.0 + jnp.exp(-a))
        h_ref[...] = (sil * b3).astype(jnp.bfloat16)


def _moe_y_kernel(te_ref, tv_ref, h_ref, w2_ref, sw_ref, y_ref):
    t = pl.program_id(0)

    @pl.when(tv_ref[t] == 1)
    def _():
        y = jnp.dot(h_ref[...], w2_ref[0].astype(jnp.bfloat16),
                    preferred_element_type=jnp.float32)
        y_ref[...] = y * sw_ref[...][:, 0:1]

    @pl.when(tv_ref[t] == 0)
    def _():
        y_ref[...] = jnp.zeros_like(y_ref)


def kernel(hidden_states, attention_mask, position_ids, ln1_w, ln2_w,
           Wq, Wk, Wv, Wo, Wg, w1, w2, w3):
    b, s, d = hidden_states.shape
    heads = _H
    hd = d // heads
    ne = Wg.shape[1]
    dff = w1.shape[2]
    scale = 1.0 / math.sqrt(hd)

    bm = min(s, 512)
    bq = min(s, 512)
    bn = min(d, 512)
    brr = min(s, 256)
    bf = 1408 if dff % 1408 == 0 else dff
    nf = dff // bf

    x0 = hidden_states.reshape(s, d)
    ln1 = ln1_w.reshape(1, d)
    ln2 = ln2_w.reshape(1, d)
    wqkv = jnp.concatenate([Wq, Wk, Wv], axis=1).astype(jnp.bfloat16)

    inv_freq = 1.0 / (10000.0 ** (jnp.arange(0, hd, 2, dtype=jnp.float32) / hd))
    t = jnp.arange(s, dtype=jnp.float32)
    freqs = jnp.outer(t, inv_freq)
    emb = jnp.concatenate([freqs, freqs], axis=-1)
    cos = jnp.cos(emb)[position_ids[0]]
    sin = jnp.sin(emb)[position_ids[0]]

    qkv = pl.pallas_call(
        _rms_matmul_kernel,
        grid=(s // bm, (3 * d) // bn),
        in_specs=[
            pl.BlockSpec((bm, d), lambda i, j: (i, 0)),
            pl.BlockSpec((1, d), lambda i, j: (0, 0)),
            pl.BlockSpec((d, bn), lambda i, j: (0, j)),
        ],
        out_specs=pl.BlockSpec((bm, bn), lambda i, j: (i, j)),
        out_shape=jax.ShapeDtypeStruct((s, 3 * d), jnp.float32),
    )(x0, ln1, wqkv)

    qkv3 = qkv.reshape(s, 3 * heads, hd).transpose(1, 0, 2)
    qk3 = qkv3[: 2 * heads]
    v3 = qkv3[2 * heads:]
    roped = pl.pallas_call(
        _rope_kernel,
        grid=(2 * heads,),
        in_specs=[
            pl.BlockSpec((1, s, hd), lambda h: (h, 0, 0)),
            pl.BlockSpec((s, hd), lambda h: (0, 0)),
            pl.BlockSpec((s, hd), lambda h: (0, 0)),
        ],
        out_specs=pl.BlockSpec((1, s, hd), lambda h: (h, 0, 0)),
        out_shape=jax.ShapeDtypeStruct((2 * heads, s, hd), jnp.float32),
    )(qk3, cos, sin)

    bk = bq
    attno = pl.pallas_call(
        functools.partial(_attn_kernel, bq=bq, bk=bk, scale=scale),
        grid=(heads, s // bq, s // bk),
        in_specs=[
            pl.BlockSpec((1, bq, hd), lambda h, iq, ik: (h, iq, 0)),
            pl.BlockSpec((1, s, hd), lambda h, iq, ik: (heads + h, 0, 0)),
            pl.BlockSpec((1, s, hd), lambda h, iq, ik: (h, 0, 0)),
        ],
        out_specs=pl.BlockSpec((1, bq, hd), lambda h, iq, ik: (h, iq, 0)),
        out_shape=jax.ShapeDtypeStruct((heads, s, hd), jnp.float32),
        scratch_shapes=[
            pltpu.VMEM((bq, hd), jnp.float32),
            pltpu.VMEM((bq, 128), jnp.float32),
            pltpu.VMEM((bq, 128), jnp.float32),
        ],
    )(roped, roped, v3)
    attno2 = attno.transpose(1, 0, 2).reshape(s, d)

    h = pl.pallas_call(
        _mm_add_kernel,
        grid=(s // bm, d // bn),
        in_specs=[
            pl.BlockSpec((bm, d), lambda i, j: (i, 0)),
            pl.BlockSpec((d, bn), lambda i, j: (0, j)),
            pl.BlockSpec((bm, bn), lambda i, j: (i, j)),
        ],
        out_specs=pl.BlockSpec((bm, bn), lambda i, j: (i, j)),
        out_shape=jax.ShapeDtypeStruct((s, d), jnp.float32),
    )(attno2, Wo, x0)

    x2 = pl.pallas_call(
        _rms_kernel,
        grid=(s // bm,),
        in_specs=[
            pl.BlockSpec((bm, d), lambda i: (i, 0)),
            pl.BlockSpec((1, d), lambda i: (0, 0)),
        ],
        out_specs=pl.BlockSpec((bm, d), lambda i: (i, 0)),
        out_shape=jax.ShapeDtypeStruct((s, d), jnp.float32),
    )(h, ln2)

    wg_pad = jnp.pad(Wg, ((0, 0), (0, 128 - ne)))
    logits_p, fullw, top_idx, top_w = pl.pallas_call(
        functools.partial(_router_kernel, e=ne),
        grid=(s // bm,),
        in_specs=[
            pl.BlockSpec((bm, d), lambda i: (i, 0)),
            pl.BlockSpec((d, 128), lambda i: (0, 0)),
        ],
        out_specs=[
            pl.BlockSpec((bm, 128), lambda i: (i, 0)),
            pl.BlockSpec((bm, 128), lambda i: (i, 0)),
            pl.BlockSpec((bm, 128), lambda i: (i, 0)),
            pl.BlockSpec((bm, 128), lambda i: (i, 0)),
        ],
        out_shape=[
            jax.ShapeDtypeStruct((s, 128), jnp.float32),
            jax.ShapeDtypeStruct((s, 128), jnp.float32),
            jax.ShapeDtypeStruct((s, 128), jnp.int32),
            jax.ShapeDtypeStruct((s, 128), jnp.float32),
        ],
    )(x2, wg_pad)

    a_n = 2 * s
    nslot = a_n + ne * _BT
    nt = nslot // _BT
    ntp = ((nt + 15) // 16) * 16
    ids_flat = top_idx[:, :2].reshape(-1)
    w_flat = top_w[:, :2].reshape(-1)
    st, sw, inva, invb, te, tv = _sc_dispatch(ids_flat, w_flat, s, nslot, ntp)

    xs = _sc_gather_rows(x2, st, nslot, d)

    grid_a = pltpu.PrefetchScalarGridSpec(
        num_scalar_prefetch=2,
        grid=(nf, nt),
        in_specs=[
            pl.BlockSpec((_BT, d), lambda f, t, te_r, tv_r: (t, 0)),
            pl.BlockSpec((1, d, bf), lambda f, t, te_r, tv_r: (te_r[t], 0, f)),
            pl.BlockSpec((1, d, bf), lambda f, t, te_r, tv_r: (te_r[t], 0, f)),
        ],
        out_specs=pl.BlockSpec((_BT, bf), lambda f, t, te_r, tv_r: (t, f)),
    )
    hbuf = pl.pallas_call(
        _moe_h_kernel, grid_spec=grid_a,
        out_shape=jax.ShapeDtypeStruct((nslot, dff), jnp.bfloat16),
    )(te, tv, xs, w1, w3)

    swb = jnp.broadcast_to(sw[:, None], (nslot, 128))
    grid_b = pltpu.PrefetchScalarGridSpec(
        num_scalar_prefetch=2,
        grid=(nt,),
        in_specs=[
            pl.BlockSpec((_BT, dff), lambda t, te_r, tv_r: (t, 0)),
            pl.BlockSpec((1, dff, d), lambda t, te_r, tv_r: (te_r[t], 0, 0)),
            pl.BlockSpec((_BT, 128), lambda t, te_r, tv_r: (t, 0)),
        ],
        out_specs=pl.BlockSpec((_BT, d), lambda t, te_r, tv_r: (t, 0)),
    )
    yw = pl.pallas_call(
        _moe_y_kernel, grid_spec=grid_b,
        out_shape=jax.ShapeDtypeStruct((nslot, d), jnp.float32),
    )(te, tv, hbuf, w2, swb)

    out2d = _sc_combine(h, yw, inva, invb, s, d)

    return (out2d.reshape(b, s, d), logits_p[:, :ne])


# speedup vs baseline: 1.1384x; 1.1093x over previous
import dataclasses
import functools
import math

import jax
import jax.numpy as jnp
from jax import lax
from jax.experimental import pallas as pl
from jax.experimental.pallas import tpu as pltpu
from jax.experimental.pallas import tpu_sc as plsc

EPS = 1e-6
NEG = -1e9
_H = 16

_BT = 128


def _sc_compiler_params():
    cp = pltpu.CompilerParams()
    if "needs_layout_passes" in pltpu.CompilerParams.__dataclass_fields__:
        cp = dataclasses.replace(cp, needs_layout_passes=False)
    return cp


def _rms_matmul_kernel(x_ref, lnw_ref, w_ref, o_ref):
    x = x_ref[...]
    var = jnp.mean(x * x, axis=-1, keepdims=True)
    xn = x * lax.rsqrt(var + EPS) * lnw_ref[...]
    o_ref[...] = jnp.dot(xn.astype(jnp.bfloat16), w_ref[...],
                         preferred_element_type=jnp.float32)


def _rope_kernel(x_ref, cos_ref, sin_ref, o_ref):
    x = x_ref[0]
    hd = x.shape[-1]
    x1 = x[:, : hd // 2]
    x2 = x[:, hd // 2:]
    rot = jnp.concatenate([-x2, x1], axis=-1)
    o_ref[0] = x * cos_ref[...] + rot * sin_ref[...]


def _attn_kernel(q_ref, k_ref, v_ref, o_ref, *, bq, scale):
    q = q_ref[0].astype(jnp.bfloat16)
    k = k_ref[0].astype(jnp.bfloat16)
    v = v_ref[0].astype(jnp.bfloat16)
    iq = pl.program_id(1)
    sc = lax.dot_general(q, k, (((1,), (1,)), ((), ())),
                         preferred_element_type=jnp.float32) * scale
    row = lax.broadcasted_iota(jnp.int32, sc.shape, 0) + iq * bq
    col = lax.broadcasted_iota(jnp.int32, sc.shape, 1)
    sc = sc + jnp.where(col > row, NEG, 0.0)
    m = jnp.max(sc, axis=1, keepdims=True)
    p = jnp.exp(sc - m)
    p = p / jnp.sum(p, axis=1, keepdims=True)
    o_ref[0] = lax.dot_general(p.astype(jnp.bfloat16), v,
                               (((1,), (0,)), ((), ())),
                               preferred_element_type=jnp.float32)


def _mm_add_kernel(a_ref, b_ref, r_ref, o_ref):
    o_ref[...] = r_ref[...] + jnp.dot(a_ref[...].astype(jnp.bfloat16),
                                      b_ref[...].astype(jnp.bfloat16),
                                      preferred_element_type=jnp.float32)


def _rms_kernel(x_ref, w_ref, o_ref):
    x = x_ref[...]
    var = jnp.mean(x * x, axis=-1, keepdims=True)
    o_ref[...] = x * lax.rsqrt(var + EPS) * w_ref[...]


def _router_kernel(x_ref, wg_ref, logits_ref, fullw_ref, idx_ref, topw_ref, *, e):
    x = x_ref[...]
    logits = jnp.dot(x, wg_ref[...], preferred_element_type=jnp.float32)
    logits_ref[...] = logits
    lane = lax.broadcasted_iota(jnp.int32, logits.shape, 1)
    valid = lane < e
    ml = jnp.where(valid, logits, NEG)
    mx = jnp.max(ml, axis=1, keepdims=True)
    ex = jnp.where(valid, jnp.exp(ml - mx), 0.0)
    probs = ex / jnp.sum(ex, axis=1, keepdims=True)
    m1 = jnp.max(probs, axis=1, keepdims=True)
    i1 = jnp.min(jnp.where(probs == m1, lane, e), axis=1, keepdims=True)
    p2 = jnp.where(lane == i1, -1.0, probs)
    m2 = jnp.max(p2, axis=1, keepdims=True)
    i2 = jnp.min(jnp.where(p2 == m2, lane, e), axis=1, keepdims=True)
    tot = m1 + m2
    w1n = m1 / tot
    w2n = m2 / tot
    fullw_ref[...] = (jnp.where(lane == i1, w1n, 0.0)
                      + jnp.where(lane == i2, w2n, 0.0))
    idx_ref[...] = jnp.where(lane == 0, i1, jnp.where(lane == 1, i2, 0))
    topw_ref[...] = jnp.where(lane == 0, w1n, jnp.where(lane == 1, w2n, 0.0))


def _sc_dispatch(ids, wflat, t_tokens, nslot, ntp):
    a_n = ids.shape[0]
    nchunk = a_n // 16
    mesh = plsc.VectorSubcoreMesh(core_axis_name="c", subcore_axis_name="s")

    @functools.partial(
        pl.kernel,
        out_type=[
            jax.ShapeDtypeStruct((nslot,), jnp.int32),
            jax.ShapeDtypeStruct((nslot,), jnp.float32),
            jax.ShapeDtypeStruct((t_tokens,), jnp.int32),
            jax.ShapeDtypeStruct((t_tokens,), jnp.int32),
            jax.ShapeDtypeStruct((ntp,), jnp.int32),
            jax.ShapeDtypeStruct((ntp,), jnp.int32),
        ],
        mesh=mesh,
        scratch_types=[
            pltpu.VMEM((a_n,), jnp.int32),
            pltpu.VMEM((a_n,), jnp.float32),
            pltpu.VMEM((nslot,), jnp.int32),
            pltpu.VMEM((nslot,), jnp.float32),
            pltpu.VMEM((t_tokens,), jnp.int32),
            pltpu.VMEM((t_tokens,), jnp.int32),
            pltpu.VMEM((ntp,), jnp.int32),
            pltpu.VMEM((ntp,), jnp.int32),
            pltpu.VMEM((16,), jnp.int32),
            pltpu.VMEM((16,), jnp.int32),
        ],
        compiler_params=_sc_compiler_params(),
    )
    def disp(ids_hbm, w_hbm, st_hbm, sw_hbm, ia_hbm, ib_hbm, te_hbm, tv_hbm,
             ids_v, w_v, st_v, sw_v, ia_v, ib_v, te_v, tv_v, ends_v, cnt_v):
        @pl.when((lax.axis_index("c") == 0) & (lax.axis_index("s") == 0))
        def _():
            pltpu.sync_copy(ids_hbm, ids_v)
            pltpu.sync_copy(w_hbm, w_v)
            iota = lax.iota(jnp.int32, 16)
            zeros16 = jnp.zeros((16,), jnp.int32)
            ends_v[...] = zeros16

            @pl.loop(0, nchunk)
            def _hist(c):
                vec = ids_v[pl.ds(c * 16, 16)]
                hv = ends_v[...]
                for e in range(8):
                    ce = jnp.sum((vec == e).astype(jnp.int32))
                    hv = hv + jnp.where(iota == e, ce, 0)
                ends_v[...] = hv

            h16 = ends_v[...]
            hp = ((h16 + (_BT - 1)) // _BT) * _BT
            ends = plsc.cumsum(hp)
            cnt_v[...] = ends - hp

            total_tiles = jnp.sum(jnp.where(iota == 7, ends, 0)) // _BT
            for c3 in range(ntp // 16):
                tid = iota + 16 * c3
                acc = jnp.zeros((16,), jnp.int32)
                for e in range(8):
                    ends_e = jnp.sum(jnp.where(iota == e, ends, 0))
                    acc = acc + (tid * _BT >= ends_e).astype(jnp.int32)
                te_v[pl.ds(16 * c3, 16)] = jnp.minimum(acc, 7)
                tv_v[pl.ds(16 * c3, 16)] = (tid < total_tiles).astype(jnp.int32)

            @pl.loop(0, nslot // 16)
            def _zero(i):
                st_v[pl.ds(i * 16, 16)] = zeros16
                sw_v[pl.ds(i * 16, 16)] = jnp.zeros((16,), jnp.float32)

            @pl.loop(0, nchunk)
            def _place(c):
                base = c * 16
                vec = ids_v[pl.ds(base, 16)]
                wv = w_v[pl.ds(base, 16)]
                cvec = cnt_v[...]
                rank = jnp.zeros((16,), jnp.int32)
                bse = jnp.zeros((16,), jnp.int32)
                for e in range(8):
                    m = vec == e
                    mi = m.astype(jnp.int32)
                    cs = plsc.cumsum(mi)
                    rank = jnp.where(m, cs - 1, rank)
                    ce = jnp.sum(jnp.where(iota == e, cvec, 0))
                    bse = jnp.where(m, ce, bse)
                    cvec = cvec + jnp.where(iota == e, jnp.sum(mi), 0)
                cnt_v[...] = cvec
                slot = bse + rank
                toks = (base + iota) // 2
                plsc.store_scatter(st_v, [slot], toks)
                plsc.store_scatter(sw_v, [slot], wv)
                evm = (iota % 2) == 0
                plsc.store_scatter(ia_v, [toks], slot, mask=evm)
                plsc.store_scatter(ib_v, [toks], slot,
                                   mask=jnp.logical_not(evm))

            pltpu.sync_copy(st_v, st_hbm)
            pltpu.sync_copy(sw_v, sw_hbm)
            pltpu.sync_copy(ia_v, ia_hbm)
            pltpu.sync_copy(ib_v, ib_hbm)
            pltpu.sync_copy(te_v, te_hbm)
            pltpu.sync_copy(tv_v, tv_hbm)

    return disp(ids, wflat)


def _sc_gather_rows(x2, sorted_tok, nslot, d):
    mesh = plsc.VectorSubcoreMesh(core_axis_name="c", subcore_axis_name="s")
    win = 32
    nworker = 32
    per = nslot // nworker

    @functools.partial(
        pl.kernel,
        out_type=jax.ShapeDtypeStruct((nslot, d), jnp.float32),
        mesh=mesh,
        scratch_types=[
            pltpu.VMEM((nslot,), jnp.int32),
            pltpu.VMEM((win, d), jnp.float32),
            pltpu.SemaphoreType.DMA,
        ],
        compiler_params=_sc_compiler_params(),
    )
    def gat(x_hbm, i_hbm, o_hbm, idx_v, rows_v, sem):
        wid = lax.axis_index("s") * 2 + lax.axis_index("c")
        pltpu.sync_copy(i_hbm, idx_v)
        base = wid * per

        @pl.loop(0, per // win)
        def _(w):
            off = base + w * win
            pltpu.async_copy(x_hbm.at[idx_v.at[pl.ds(off, win)]],
                             rows_v, sem).wait()
            pltpu.sync_copy(rows_v, o_hbm.at[pl.ds(off, win)])

    return gat(x2, sorted_tok)


def _sc_combine(h, yw, inva, invb, t_tokens, d):
    mesh = plsc.VectorSubcoreMesh(core_axis_name="c", subcore_axis_name="s")
    win = 16
    nworker = 32
    per = t_tokens // nworker
    nw = per // win

    @functools.partial(
        pl.kernel,
        out_type=jax.ShapeDtypeStruct((t_tokens, d), jnp.float32),
        mesh=mesh,
        scratch_types=[
            pltpu.VMEM((per,), jnp.int32),
            pltpu.VMEM((per,), jnp.int32),
            pltpu.VMEM((2, win, d), jnp.float32),
            pltpu.VMEM((2, win, d), jnp.float32),
            pltpu.VMEM((win, d), jnp.float32),
            pltpu.SemaphoreType.DMA,
            pltpu.SemaphoreType.DMA,
        ],
        compiler_params=_sc_compiler_params(),
    )
    def comb(yw_hbm, ia_hbm, ib_hbm, h_hbm, o_hbm,
             ia_v, ib_v, ya_s, yb_s, hb_s, sem0, sem1):
        wid = lax.axis_index("s") * 2 + lax.axis_index("c")
        t0 = wid * per
        pltpu.sync_copy(ia_hbm.at[pl.ds(t0, per)], ia_v)
        pltpu.sync_copy(ib_hbm.at[pl.ds(t0, per)], ib_v)
        sems = (sem0, sem1)

        def start(w, buf):
            pltpu.async_copy(yw_hbm.at[ia_v.at[pl.ds(w * win, win)]],
                             ya_s.at[buf], sems[buf])
            pltpu.async_copy(yw_hbm.at[ib_v.at[pl.ds(w * win, win)]],
                             yb_s.at[buf], sems[buf])

        def finish(w, buf):
            pltpu.make_async_copy(yw_hbm.at[ia_v.at[pl.ds(w * win, win)]],
                                  ya_s.at[buf], sems[buf]).wait()
            pltpu.make_async_copy(yw_hbm.at[ib_v.at[pl.ds(w * win, win)]],
                                  yb_s.at[buf], sems[buf]).wait()

        start(0, 0)
        for w in range(nw):
            buf = w % 2
            if w + 1 < nw:
                start(w + 1, 1 - buf)
            tb = t0 + w * win
            pltpu.sync_copy(h_hbm.at[pl.ds(tb, win)], hb_s)
            finish(w, buf)

            @pl.loop(0, win)
            def _rows(r):
                for cc in range(d // 16):
                    sl = pl.ds(cc * 16, 16)
                    hb_s[r, sl] = (hb_s[r, sl] + ya_s[buf, r, sl]
                                   + yb_s[buf, r, sl])

            pltpu.sync_copy(hb_s, o_hbm.at[pl.ds(tb, win)])

    return comb(yw, inva, invb, h)


def _moe_h_kernel(te_ref, tv_ref, x_ref, w1_ref, w3_ref, h_ref):
    t = pl.program_id(1)

    @pl.when(tv_ref[t] == 1)
    def _():
        x = x_ref[...].astype(jnp.bfloat16)
        a = jnp.dot(x, w1_ref[0].astype(jnp.bfloat16),
                    preferred_element_type=jnp.float32)
        b3 = jnp.dot(x, w3_ref[0].astype(jnp.bfloat16),
                     preferred_element_type=jnp.float32)
        sil = a / (1.0 + jnp.exp(-a))
        h_ref[...] = (sil * b3).astype(jnp.bfloat16)


def _moe_y_kernel(te_ref, tv_ref, h_ref, w2_ref, sw_ref, y_ref):
    t = pl.program_id(0)

    @pl.when(tv_ref[t] == 1)
    def _():
        y = jnp.dot(h_ref[...], w2_ref[0].astype(jnp.bfloat16),
                    preferred_element_type=jnp.float32)
        y_ref[...] = y * sw_ref[...][:, 0:1]

    @pl.when(tv_ref[t] == 0)
    def _():
        y_ref[...] = jnp.zeros_like(y_ref)


def kernel(hidden_states, attention_mask, position_ids, ln1_w, ln2_w,
           Wq, Wk, Wv, Wo, Wg, w1, w2, w3):
    b, s, d = hidden_states.shape
    heads = _H
    hd = d // heads
    ne = Wg.shape[1]
    dff = w1.shape[2]
    scale = 1.0 / math.sqrt(hd)

    bm = min(s, 512)
    bq = min(s, 512)
    bn = min(d, 512)
    brr = min(s, 256)
    bf = 1408 if dff % 1408 == 0 else dff
    nf = dff // bf

    x0 = hidden_states.reshape(s, d)
    ln1 = ln1_w.reshape(1, d)
    ln2 = ln2_w.reshape(1, d)
    wqkv = jnp.concatenate([Wq, Wk, Wv], axis=1).astype(jnp.bfloat16)

    inv_freq = 1.0 / (10000.0 ** (jnp.arange(0, hd, 2, dtype=jnp.float32) / hd))
    t = jnp.arange(s, dtype=jnp.float32)
    freqs = jnp.outer(t, inv_freq)
    emb = jnp.concatenate([freqs, freqs], axis=-1)
    cos = jnp.cos(emb)[position_ids[0]]
    sin = jnp.sin(emb)[position_ids[0]]

    qkv = pl.pallas_call(
        _rms_matmul_kernel,
        grid=(s // bm, (3 * d) // bn),
        in_specs=[
            pl.BlockSpec((bm, d), lambda i, j: (i, 0)),
            pl.BlockSpec((1, d), lambda i, j: (0, 0)),
            pl.BlockSpec((d, bn), lambda i, j: (0, j)),
        ],
        out_specs=pl.BlockSpec((bm, bn), lambda i, j: (i, j)),
        out_shape=jax.ShapeDtypeStruct((s, 3 * d), jnp.float32),
    )(x0, ln1, wqkv)

    qkv3 = qkv.reshape(s, 3 * heads, hd).transpose(1, 0, 2)
    qk3 = qkv3[: 2 * heads]
    v3 = qkv3[2 * heads:]
    roped = pl.pallas_call(
        _rope_kernel,
        grid=(2 * heads, s // bm),
        in_specs=[
            pl.BlockSpec((1, bm, hd), lambda h, i: (h, i, 0)),
            pl.BlockSpec((bm, hd), lambda h, i: (i, 0)),
            pl.BlockSpec((bm, hd), lambda h, i: (i, 0)),
        ],
        out_specs=pl.BlockSpec((1, bm, hd), lambda h, i: (h, i, 0)),
        out_shape=jax.ShapeDtypeStruct((2 * heads, s, hd), jnp.float32),
    )(qk3, cos, sin)

    attno = pl.pallas_call(
        functools.partial(_attn_kernel, bq=bq, scale=scale),
        grid=(heads, s // bq),
        in_specs=[
            pl.BlockSpec((1, bq, hd), lambda h, iq: (h, iq, 0)),
            pl.BlockSpec((1, s, hd), lambda h, iq: (heads + h, 0, 0)),
            pl.BlockSpec((1, s, hd), lambda h, iq: (h, 0, 0)),
        ],
        out_specs=pl.BlockSpec((1, bq, hd), lambda h, iq: (h, iq, 0)),
        out_shape=jax.ShapeDtypeStruct((heads, s, hd), jnp.float32),
    )(roped, roped, v3)
    attno2 = attno.transpose(1, 0, 2).reshape(s, d)

    h = pl.pallas_call(
        _mm_add_kernel,
        grid=(s // bm, d // bn),
        in_specs=[
            pl.BlockSpec((bm, d), lambda i, j: (i, 0)),
            pl.BlockSpec((d, bn), lambda i, j: (0, j)),
            pl.BlockSpec((bm, bn), lambda i, j: (i, j)),
        ],
        out_specs=pl.BlockSpec((bm, bn), lambda i, j: (i, j)),
        out_shape=jax.ShapeDtypeStruct((s, d), jnp.float32),
    )(attno2, Wo, x0)

    x2 = pl.pallas_call(
        _rms_kernel,
        grid=(s // bm,),
        in_specs=[
            pl.BlockSpec((bm, d), lambda i: (i, 0)),
            pl.BlockSpec((1, d), lambda i: (0, 0)),
        ],
        out_specs=pl.BlockSpec((bm, d), lambda i: (i, 0)),
        out_shape=jax.ShapeDtypeStruct((s, d), jnp.float32),
    )(h, ln2)

    wg_pad = jnp.pad(Wg, ((0, 0), (0, 128 - ne)))
    logits_p, fullw, top_idx, top_w = pl.pallas_call(
        functools.partial(_router_kernel, e=ne),
        grid=(s // bm,),
        in_specs=[
            pl.BlockSpec((bm, d), lambda i: (i, 0)),
            pl.BlockSpec((d, 128), lambda i: (0, 0)),
        ],
        out_specs=[
            pl.BlockSpec((bm, 128), lambda i: (i, 0)),
            pl.BlockSpec((bm, 128), lambda i: (i, 0)),
            pl.BlockSpec((bm, 128), lambda i: (i, 0)),
            pl.BlockSpec((bm, 128), lambda i: (i, 0)),
        ],
        out_shape=[
            jax.ShapeDtypeStruct((s, 128), jnp.float32),
            jax.ShapeDtypeStruct((s, 128), jnp.float32),
            jax.ShapeDtypeStruct((s, 128), jnp.int32),
            jax.ShapeDtypeStruct((s, 128), jnp.float32),
        ],
    )(x2, wg_pad)

    a_n = 2 * s
    nslot = a_n + ne * _BT
    nt = nslot // _BT
    ntp = ((nt + 15) // 16) * 16
    ids_flat = top_idx[:, :2].reshape(-1)
    w_flat = top_w[:, :2].reshape(-1)
    st, sw, inva, invb, te, tv = _sc_dispatch(ids_flat, w_flat, s, nslot, ntp)

    xs = _sc_gather_rows(x2, st, nslot, d)

    grid_a = pltpu.PrefetchScalarGridSpec(
        num_scalar_prefetch=2,
        grid=(nf, nt),
        in_specs=[
            pl.BlockSpec((_BT, d), lambda f, t, te_r, tv_r: (t, 0)),
            pl.BlockSpec((1, d, bf), lambda f, t, te_r, tv_r: (te_r[t], 0, f)),
            pl.BlockSpec((1, d, bf), lambda f, t, te_r, tv_r: (te_r[t], 0, f)),
        ],
        out_specs=pl.BlockSpec((_BT, bf), lambda f, t, te_r, tv_r: (t, f)),
    )
    hbuf = pl.pallas_call(
        _moe_h_kernel, grid_spec=grid_a,
        out_shape=jax.ShapeDtypeStruct((nslot, dff), jnp.bfloat16),
    )(te, tv, xs, w1, w3)

    swb = jnp.broadcast_to(sw[:, None], (nslot, 128))
    grid_b = pltpu.PrefetchScalarGridSpec(
        num_scalar_prefetch=2,
        grid=(nt,),
        in_specs=[
            pl.BlockSpec((_BT, dff), lambda t, te_r, tv_r: (t, 0)),
            pl.BlockSpec((1, dff, d), lambda t, te_r, tv_r: (te_r[t], 0, 0)),
            pl.BlockSpec((_BT, 128), lambda t, te_r, tv_r: (t, 0)),
        ],
        out_specs=pl.BlockSpec((_BT, d), lambda t, te_r, tv_r: (t, 0)),
    )
    yw = pl.pallas_call(
        _moe_y_kernel, grid_spec=grid_b,
        out_shape=jax.ShapeDtypeStruct((nslot, d), jnp.float32),
    )(te, tv, hbuf, w2, swb)

    out2d = _sc_combine(h, yw, inva, invb, s, d)

    return (out2d.reshape(b, s, d), logits_p[:, :ne])
